# baseline (device time: 47235 ns/iter reference)
import jax
import jax.numpy as jnp
from jax import lax
from jax.experimental import pallas as pl
from jax.experimental.pallas import tpu as pltpu

N_DEV = 4


def kernel(x, W1, W2):
    m, _ = x.shape
    n = W2.shape[1]
    CH = m // N_DEV

    def body(x_ref, w1_ref, w2_ref, out_ref, red_ref, stage_ref,
             send_sems, recv_sems):
        my = lax.axis_index("i")
        left = (my - 1) % N_DEV
        right = (my + 1) % N_DEV
        diag = (my + 2) % N_DEV


        def compute_chunk(c):
            rows = pl.ds(c * CH, CH)
            h = jnp.maximum(
                jnp.dot(x_ref[rows, :], w1_ref[...],
                        preferred_element_type=jnp.float32),
                0.0,
            )
            return jnp.dot(h, w2_ref[...], preferred_element_type=jnp.float32)

        def rs_send(c, slot, target):
            red_ref[pl.ds(c * CH, CH), :] = compute_chunk(c).astype(jnp.bfloat16)
            rdma = pltpu.make_async_remote_copy(
                src_ref=red_ref.at[pl.ds(c * CH, CH), :],
                dst_ref=stage_ref.at[slot],
                send_sem=send_sems.at[0, slot],
                recv_sem=recv_sems.at[0, slot],
                device_id=(target,),
                device_id_type=pl.DeviceIdType.MESH,
            )
            rdma.start()
            return rdma

        rd = rs_send(diag, 2, diag)
        rr = rs_send(right, 0, right)
        rl = rs_send(left, 1, left)
        p_own = compute_chunk(my)

        rd.wait()
        rr.wait()
        rl.wait()

        rows_my = pl.ds(my * CH, CH)
        t = (p_own
             + stage_ref[0].astype(jnp.float32)
             + stage_ref[1].astype(jnp.float32)
             + stage_ref[2].astype(jnp.float32))
        out_ref[rows_my, :] = t
        red_ref[rows_my, :] = t.astype(jnp.bfloat16)

        def ag_send(slot, target):
            rdma = pltpu.make_async_remote_copy(
                src_ref=red_ref.at[rows_my, :],
                dst_ref=red_ref.at[rows_my, :],
                send_sem=send_sems.at[1, slot],
                recv_sem=recv_sems.at[1, slot],
                device_id=(target,),
                device_id_type=pl.DeviceIdType.MESH,
            )
            rdma.start()
            return rdma

        ad = ag_send(2, diag)
        ar = ag_send(0, right)
        al = ag_send(1, left)

        def cast_chunk(c):
            rows = pl.ds(c * CH, CH)
            out_ref[rows, :] = red_ref[rows, :].astype(jnp.float32)

        ar.wait()
        cast_chunk(left)
        al.wait()
        cast_chunk(right)
        ad.wait()
        cast_chunk(diag)

    return pl.pallas_call(
        body,
        out_shape=jax.ShapeDtypeStruct((m, n), jnp.float32),
        in_specs=[pl.BlockSpec(memory_space=pltpu.VMEM)] * 3,
        out_specs=pl.BlockSpec(memory_space=pltpu.VMEM),
        scratch_shapes=[
            pltpu.VMEM((m, n), jnp.bfloat16),
            pltpu.VMEM((3, CH, n), jnp.bfloat16),
            pltpu.SemaphoreType.DMA((2, 3)),
            pltpu.SemaphoreType.DMA((2, 3)),
        ],
        compiler_params=pltpu.CompilerParams(),
    )(x, W1, W2)


# device time: 45048 ns/iter; 1.0485x vs baseline; 1.0485x over previous
import jax
import jax.numpy as jnp
from jax import lax
from jax.experimental import pallas as pl
from jax.experimental.pallas import tpu as pltpu

N_DEV = 4


def kernel(x, W1, W2):
    m, _ = x.shape
    n = W2.shape[1]
    CH = m // N_DEV

    def body(x_ref, w1_ref, w2_ref, out_ref, red_ref, stage_ref,
             send_sems, recv_sems):
        my = lax.axis_index("i")
        left = (my - 1) % N_DEV
        right = (my + 1) % N_DEV
        diag = (my + 2) % N_DEV

        barrier_sem = pltpu.get_barrier_semaphore()
        for _round in range(2):
            for nbr in (left, right):
                pl.semaphore_signal(
                    barrier_sem, inc=1,
                    device_id=(nbr,), device_id_type=pl.DeviceIdType.MESH,
                )
            pl.semaphore_wait(barrier_sem, 2)

        def compute_chunk(c):
            rows = pl.ds(c * CH, CH)
            h = jnp.maximum(
                jnp.dot(x_ref[rows, :], w1_ref[...],
                        preferred_element_type=jnp.float32),
                0.0,
            )
            return jnp.dot(h, w2_ref[...], preferred_element_type=jnp.float32)

        def rs_send(c, slot, target):
            red_ref[pl.ds(c * CH, CH), :] = compute_chunk(c).astype(jnp.bfloat16)
            rdma = pltpu.make_async_remote_copy(
                src_ref=red_ref.at[pl.ds(c * CH, CH), :],
                dst_ref=stage_ref.at[slot],
                send_sem=send_sems.at[0, slot],
                recv_sem=recv_sems.at[0, slot],
                device_id=(target,),
                device_id_type=pl.DeviceIdType.MESH,
            )
            rdma.start()
            return rdma

        rd = rs_send(diag, 2, diag)
        rr = rs_send(right, 0, right)
        rl = rs_send(left, 1, left)
        p_own = compute_chunk(my)

        rd.wait()
        rr.wait()
        rl.wait()

        rows_my = pl.ds(my * CH, CH)
        t = (p_own
             + stage_ref[0].astype(jnp.float32)
             + stage_ref[1].astype(jnp.float32)
             + stage_ref[2].astype(jnp.float32))
        out_ref[rows_my, :] = t
        red_ref[rows_my, :] = t.astype(jnp.bfloat16)

        def ag_send(slot, target):
            rdma = pltpu.make_async_remote_copy(
                src_ref=red_ref.at[rows_my, :],
                dst_ref=red_ref.at[rows_my, :],
                send_sem=send_sems.at[1, slot],
                recv_sem=recv_sems.at[1, slot],
                device_id=(target,),
                device_id_type=pl.DeviceIdType.MESH,
            )
            rdma.start()
            return rdma

        ad = ag_send(2, diag)
        ar = ag_send(0, right)
        al = ag_send(1, left)

        def cast_chunk(c):
            rows = pl.ds(c * CH, CH)
            out_ref[rows, :] = red_ref[rows, :].astype(jnp.float32)

        ar.wait()
        cast_chunk(left)
        al.wait()
        cast_chunk(right)
        ad.wait()
        cast_chunk(diag)

    return pl.pallas_call(
        body,
        out_shape=jax.ShapeDtypeStruct((m, n), jnp.float32),
        in_specs=[pl.BlockSpec(memory_space=pltpu.VMEM)] * 3,
        out_specs=pl.BlockSpec(memory_space=pltpu.VMEM),
        scratch_shapes=[
            pltpu.VMEM((m, n), jnp.bfloat16),
            pltpu.VMEM((3, CH, n), jnp.bfloat16),
            pltpu.SemaphoreType.DMA((2, 3)),
            pltpu.SemaphoreType.DMA((2, 3)),
        ],
        compiler_params=pltpu.CompilerParams(collective_id=0),
    )(x, W1, W2)


# device time: 40237 ns/iter; 1.1739x vs baseline; 1.1196x over previous
import jax
import jax.numpy as jnp
from jax import lax
from jax.experimental import pallas as pl
from jax.experimental.pallas import tpu as pltpu

N_DEV = 4


def kernel(x, W1, W2):
    m, _ = x.shape
    n = W2.shape[1]
    CH = m // N_DEV
    SUB = CH // 2
    HALF = n // 2

    def body(x_ref, w1_ref, w2_ref, out_ref, red_ref, stage_ref, stage2_ref,
             send01, recv01, send2, recv2, send_ag, recv_ag):
        my = lax.axis_index("i")
        left = (my - 1) % N_DEV
        right = (my + 1) % N_DEV

        barrier_sem = pltpu.get_barrier_semaphore()
        for nbr in (left, right):
            pl.semaphore_signal(
                barrier_sem, inc=1,
                device_id=(nbr,), device_id_type=pl.DeviceIdType.MESH,
            )
        pl.semaphore_wait(barrier_sem, 2)

        def compute_chunk(c):
            rows = pl.ds(c * CH, CH)
            h = jnp.maximum(
                jnp.dot(x_ref[rows, :], w1_ref[...],
                        preferred_element_type=jnp.float32),
                0.0,
            )
            red_ref[rows, :] = jnp.dot(
                h, w2_ref[...], preferred_element_type=jnp.float32
            ).astype(jnp.bfloat16)

        def rs_rdma(s):
            cp = (my - s) % N_DEV
            cm = (my + s) % N_DEV
            rp = pltpu.make_async_remote_copy(
                src_ref=red_ref.at[pl.ds(cp * CH, CH), pl.ds(0, HALF)],
                dst_ref=stage_ref.at[0, s],
                send_sem=send01.at[0, s],
                recv_sem=recv01.at[0, s],
                device_id=(right,),
                device_id_type=pl.DeviceIdType.MESH,
            )
            rm = pltpu.make_async_remote_copy(
                src_ref=red_ref.at[pl.ds(cm * CH, CH), pl.ds(HALF, HALF)],
                dst_ref=stage_ref.at[1, s],
                send_sem=send01.at[1, s],
                recv_sem=recv01.at[1, s],
                device_id=(left,),
                device_id_type=pl.DeviceIdType.MESH,
            )
            rp.start()
            rm.start()
            return rp, rm

        def rs_accum(s, rp, rm):
            rp.wait()
            rm.wait()
            cpr = (my - s - 1) % N_DEV
            cmr = (my + s + 1) % N_DEV
            red_ref[pl.ds(cpr * CH, CH), 0:HALF] += stage_ref[0, s]
            red_ref[pl.ds(cmr * CH, CH), HALF:n] += stage_ref[1, s]

        def cast_half(c, lo):
            rows = pl.ds(c * CH, CH)
            out_ref[rows, lo:lo + HALF] = (
                red_ref[rows, lo:lo + HALF].astype(jnp.float32))

        compute_chunk(my)
        rp0, rm0 = rs_rdma(0)
        compute_chunk((my - 1) % N_DEV)
        compute_chunk((my + 1) % N_DEV)
        rs_accum(0, rp0, rm0)
        rp1, rm1 = rs_rdma(1)
        compute_chunk((my + 2) % N_DEV)
        rs_accum(1, rp1, rm1)

        c2 = (my + 2) % N_DEV

        def rs2_rdma(j):
            rows = pl.ds(c2 * CH + j * SUB, SUB)
            rp = pltpu.make_async_remote_copy(
                src_ref=red_ref.at[rows, pl.ds(0, HALF)],
                dst_ref=stage2_ref.at[0, j],
                send_sem=send2.at[0, j],
                recv_sem=recv2.at[0, j],
                device_id=(right,),
                device_id_type=pl.DeviceIdType.MESH,
            )
            rm = pltpu.make_async_remote_copy(
                src_ref=red_ref.at[rows, pl.ds(HALF, HALF)],
                dst_ref=stage2_ref.at[1, j],
                send_sem=send2.at[1, j],
                recv_sem=recv2.at[1, j],
                device_id=(left,),
                device_id_type=pl.DeviceIdType.MESH,
            )
            rp.start()
            rm.start()
            return rp, rm

        r2 = [rs2_rdma(j) for j in range(2)]

        def ag_rdma(s, j):
            cp = (my + 1 - s) % N_DEV
            cm = (my - 1 + s) % N_DEV
            rows_p = pl.ds(cp * CH + j * SUB, SUB)
            rows_m = pl.ds(cm * CH + j * SUB, SUB)
            ap = pltpu.make_async_remote_copy(
                src_ref=red_ref.at[rows_p, pl.ds(0, HALF)],
                dst_ref=red_ref.at[rows_p, pl.ds(0, HALF)],
                send_sem=send_ag.at[0, s, j],
                recv_sem=recv_ag.at[0, s, j],
                device_id=(right,),
                device_id_type=pl.DeviceIdType.MESH,
            )
            am = pltpu.make_async_remote_copy(
                src_ref=red_ref.at[rows_m, pl.ds(HALF, HALF)],
                dst_ref=red_ref.at[rows_m, pl.ds(HALF, HALF)],
                send_sem=send_ag.at[1, s, j],
                recv_sem=recv_ag.at[1, s, j],
                device_id=(left,),
                device_id_type=pl.DeviceIdType.MESH,
            )
            ap.start()
            am.start()
            return ap, am

        cp2 = (my + 1) % N_DEV
        cm2 = (my - 1) % N_DEV
        ag = [[None, None], [None, None], [None, None]]
        for j in range(2):
            rp, rm = r2[j]
            rp.wait()
            rm.wait()
            red_ref[pl.ds(cp2 * CH + j * SUB, SUB), 0:HALF] += stage2_ref[0, j]
            red_ref[pl.ds(cm2 * CH + j * SUB, SUB), HALF:n] += stage2_ref[1, j]
            ag[0][j] = ag_rdma(0, j)
        cast_half(cp2, 0)
        cast_half(cm2, HALF)
        for s in range(3):
            for j in range(2):
                ap, am = ag[s][j]
                ap.wait()
                am.wait()
                if s < 2:
                    ag[s + 1][j] = ag_rdma(s + 1, j)
            cast_half((my - s) % N_DEV, 0)
            cast_half((my + s) % N_DEV, HALF)

    return pl.pallas_call(
        body,
        out_shape=jax.ShapeDtypeStruct((m, n), jnp.float32),
        in_specs=[pl.BlockSpec(memory_space=pltpu.VMEM)] * 3,
        out_specs=pl.BlockSpec(memory_space=pltpu.VMEM),
        scratch_shapes=[
            pltpu.VMEM((m, n), jnp.bfloat16),
            pltpu.VMEM((2, 2, CH, HALF), jnp.bfloat16),
            pltpu.VMEM((2, 2, SUB, HALF), jnp.bfloat16),
            pltpu.SemaphoreType.DMA((2, 2)),
            pltpu.SemaphoreType.DMA((2, 2)),
            pltpu.SemaphoreType.DMA((2, 2)),
            pltpu.SemaphoreType.DMA((2, 2)),
            pltpu.SemaphoreType.DMA((2, 3, 2)),
            pltpu.SemaphoreType.DMA((2, 3, 2)),
        ],
        compiler_params=pltpu.CompilerParams(collective_id=0),
    )(x, W1, W2)


# device time: 38963 ns/iter; 1.2123x vs baseline; 1.0327x over previous
import jax
import jax.numpy as jnp
from jax import lax
from jax.experimental import pallas as pl
from jax.experimental.pallas import tpu as pltpu

N_DEV = 4


def kernel(x, W1, W2):
    m, _ = x.shape
    n = W2.shape[1]
    CH = m // N_DEV
    NSUB = 4
    SUB = CH // NSUB
    HALF = n // 2

    def body(x_ref, w1_ref, w2_ref, out_ref, red_ref, stage_ref, stage2_ref,
             send01, recv01, send2, recv2, send_ag, recv_ag):
        my = lax.axis_index("i")
        left = (my - 1) % N_DEV
        right = (my + 1) % N_DEV

        barrier_sem = pltpu.get_barrier_semaphore()
        for nbr in (left, right):
            pl.semaphore_signal(
                barrier_sem, inc=1,
                device_id=(nbr,), device_id_type=pl.DeviceIdType.MESH,
            )
        pl.semaphore_wait(barrier_sem, 2)

        def compute_chunk(c):
            rows = pl.ds(c * CH, CH)
            h = jnp.maximum(
                jnp.dot(x_ref[rows, :], w1_ref[...],
                        preferred_element_type=jnp.float32),
                0.0,
            )
            red_ref[rows, :] = jnp.dot(
                h, w2_ref[...], preferred_element_type=jnp.float32
            ).astype(jnp.bfloat16)

        def rs_rdma(s):
            cp = (my - s) % N_DEV
            cm = (my + s) % N_DEV
            rp = pltpu.make_async_remote_copy(
                src_ref=red_ref.at[pl.ds(cp * CH, CH), pl.ds(0, HALF)],
                dst_ref=stage_ref.at[0, s],
                send_sem=send01.at[0, s],
                recv_sem=recv01.at[0, s],
                device_id=(right,),
                device_id_type=pl.DeviceIdType.MESH,
            )
            rm = pltpu.make_async_remote_copy(
                src_ref=red_ref.at[pl.ds(cm * CH, CH), pl.ds(HALF, HALF)],
                dst_ref=stage_ref.at[1, s],
                send_sem=send01.at[1, s],
                recv_sem=recv01.at[1, s],
                device_id=(left,),
                device_id_type=pl.DeviceIdType.MESH,
            )
            rp.start()
            rm.start()
            return rp, rm

        def rs_accum(s, rp, rm):
            rp.wait()
            rm.wait()
            cpr = (my - s - 1) % N_DEV
            cmr = (my + s + 1) % N_DEV
            red_ref[pl.ds(cpr * CH, CH), 0:HALF] += stage_ref[0, s]
            red_ref[pl.ds(cmr * CH, CH), HALF:n] += stage_ref[1, s]

        def cast_half(c, lo):
            rows = pl.ds(c * CH, CH)
            out_ref[rows, lo:lo + HALF] = (
                red_ref[rows, lo:lo + HALF].astype(jnp.float32))

        compute_chunk(my)
        rp0, rm0 = rs_rdma(0)
        compute_chunk((my - 1) % N_DEV)
        compute_chunk((my + 1) % N_DEV)
        rs_accum(0, rp0, rm0)
        rp1, rm1 = rs_rdma(1)
        compute_chunk((my + 2) % N_DEV)
        rs_accum(1, rp1, rm1)

        c2 = (my + 2) % N_DEV

        def rs2_rdma(j):
            rows = pl.ds(c2 * CH + j * SUB, SUB)
            rp = pltpu.make_async_remote_copy(
                src_ref=red_ref.at[rows, pl.ds(0, HALF)],
                dst_ref=stage2_ref.at[0, j],
                send_sem=send2.at[0, j],
                recv_sem=recv2.at[0, j],
                device_id=(right,),
                device_id_type=pl.DeviceIdType.MESH,
            )
            rm = pltpu.make_async_remote_copy(
                src_ref=red_ref.at[rows, pl.ds(HALF, HALF)],
                dst_ref=stage2_ref.at[1, j],
                send_sem=send2.at[1, j],
                recv_sem=recv2.at[1, j],
                device_id=(left,),
                device_id_type=pl.DeviceIdType.MESH,
            )
            rp.start()
            rm.start()
            return rp, rm

        r2 = [rs2_rdma(j) for j in range(NSUB)]

        def ag_rdma(s, j):
            cp = (my + 1 - s) % N_DEV
            cm = (my - 1 + s) % N_DEV
            rows_p = pl.ds(cp * CH + j * SUB, SUB)
            rows_m = pl.ds(cm * CH + j * SUB, SUB)
            ap = pltpu.make_async_remote_copy(
                src_ref=red_ref.at[rows_p, pl.ds(0, HALF)],
                dst_ref=red_ref.at[rows_p, pl.ds(0, HALF)],
                send_sem=send_ag.at[0, s, j],
                recv_sem=recv_ag.at[0, s, j],
                device_id=(right,),
                device_id_type=pl.DeviceIdType.MESH,
            )
            am = pltpu.make_async_remote_copy(
                src_ref=red_ref.at[rows_m, pl.ds(HALF, HALF)],
                dst_ref=red_ref.at[rows_m, pl.ds(HALF, HALF)],
                send_sem=send_ag.at[1, s, j],
                recv_sem=recv_ag.at[1, s, j],
                device_id=(left,),
                device_id_type=pl.DeviceIdType.MESH,
            )
            ap.start()
            am.start()
            return ap, am

        cp2 = (my + 1) % N_DEV
        cm2 = (my - 1) % N_DEV
        ag = [[None] * NSUB for _ in range(3)]
        for j in range(NSUB):
            rp, rm = r2[j]
            rp.wait()
            rm.wait()
            red_ref[pl.ds(cp2 * CH + j * SUB, SUB), 0:HALF] += stage2_ref[0, j]
            red_ref[pl.ds(cm2 * CH + j * SUB, SUB), HALF:n] += stage2_ref[1, j]
            ag[0][j] = ag_rdma(0, j)
        cast_half(cp2, 0)
        cast_half(cm2, HALF)
        for s in range(3):
            for j in range(NSUB):
                ap, am = ag[s][j]
                ap.wait()
                am.wait()
                if s < 2:
                    ag[s + 1][j] = ag_rdma(s + 1, j)
            cast_half((my - s) % N_DEV, 0)
            cast_half((my + s) % N_DEV, HALF)

    return pl.pallas_call(
        body,
        out_shape=jax.ShapeDtypeStruct((m, n), jnp.float32),
        in_specs=[pl.BlockSpec(memory_space=pltpu.VMEM)] * 3,
        out_specs=pl.BlockSpec(memory_space=pltpu.VMEM),
        scratch_shapes=[
            pltpu.VMEM((m, n), jnp.bfloat16),
            pltpu.VMEM((2, 2, CH, HALF), jnp.bfloat16),
            pltpu.VMEM((2, 4, SUB, HALF), jnp.bfloat16),
            pltpu.SemaphoreType.DMA((2, 2)),
            pltpu.SemaphoreType.DMA((2, 2)),
            pltpu.SemaphoreType.DMA((2, 4)),
            pltpu.SemaphoreType.DMA((2, 4)),
            pltpu.SemaphoreType.DMA((2, 3, 4)),
            pltpu.SemaphoreType.DMA((2, 3, 4)),
        ],
        compiler_params=pltpu.CompilerParams(collective_id=0),
    )(x, W1, W2)
